# Initial kernel scaffold; baseline (speedup 1.0000x reference)
import jax
import jax.numpy as jnp
from jax.experimental import pallas as pl


def kernel(xyz, centers, H4, H8, H12, W1, b1, g1, be1, W2, b2, g2, be2):
    raise NotImplementedError("write your pallas kernel here")


# speedup vs baseline: 6.4106x; 6.4106x over previous
import functools

import jax
import jax.numpy as jnp
from jax import lax
from jax.experimental import pallas as pl
from jax.experimental.pallas import tpu as pltpu
from jax.experimental.pallas import tpu_sc as plsc

B, N, G, D = 16, 2048, 128, 512
C = 512
NB_BLK = 256
NSTEPS = (B * N) // NB_BLK
NBB = N // NB_BLK

SC_NC, SC_NS, SC_L = 2, 16, 16
NW = SC_NC * SC_NS
PTS_W = (B * N) // NW
P_CHUNK = 32
N_CHUNKS = PTS_W // P_CHUNK


def _k1a_body(h4, h8, h12, w1a, w1b, w1c, b1, f1):
    acc = jnp.dot(h4[...], w1a[...], preferred_element_type=jnp.float32)
    acc += jnp.dot(h8[...], w1b[...], preferred_element_type=jnp.float32)
    acc += jnp.dot(h12[...], w1c[...], preferred_element_type=jnp.float32)
    f1[...] = acc + b1[...]


def _k1a(h4, h8, h12, w1a, w1b, w1c, b1):
    grid = ((B * G) // NB_BLK,)
    blk = pl.BlockSpec((NB_BLK, D), lambda i: (i, 0))
    wblk = pl.BlockSpec((D, C), lambda i: (0, 0))
    return pl.pallas_call(
        _k1a_body,
        grid=grid,
        in_specs=[blk, blk, blk, wblk, wblk, wblk,
                  pl.BlockSpec((1, C), lambda i: (0, 0))],
        out_specs=pl.BlockSpec((NB_BLK, C), lambda i: (i, 0)),
        out_shape=jax.ShapeDtypeStruct((B * G, C), jnp.float32),
        compiler_params=pltpu.CompilerParams(
            dimension_semantics=("arbitrary",)),
    )(h4, h8, h12, w1a, w1b, w1c, b1)


def _k1b_body(xyz, cen, idx_out, w_out, m_out):
    nb = pl.program_id(1)
    x = xyz[0]
    c = cen[0]
    cg2 = jnp.sum(c * c, axis=1, keepdims=True)
    ones_row = jnp.ones((1, 8), jnp.float32)
    xn2 = lax.dot_general(ones_row, x * x,
                          (((1,), (1,)), ((), ())),
                          preferred_element_type=jnp.float32,
                          precision=lax.Precision.HIGHEST)
    cross = lax.dot_general(c.astype(jnp.bfloat16), x.astype(jnp.bfloat16),
                            (((1,), (1,)), ((), ())),
                            preferred_element_type=jnp.float32)
    d = cg2 - 2.0 * cross + xn2
    iota_g = lax.broadcasted_iota(jnp.int32, (G, NB_BLK), 0)
    sels, mins = [], []
    for _ in range(3):
        m = jnp.min(d, axis=0, keepdims=True)
        cand = jnp.where(d == m, iota_g, G)
        sel = jnp.min(cand, axis=0, keepdims=True)
        oh = iota_g == sel
        d = jnp.where(oh, jnp.inf, d)
        sels.append(sel)
        mins.append(m)
    r0 = 1.0 / (mins[0] + 1e-8)
    r1 = 1.0 / (mins[1] + 1e-8)
    r2 = 1.0 / (mins[2] + 1e-8)
    rs = r0 + r1 + r2
    w0, w1, w2 = r0 / rs, r1 / rs, r2 / rs
    ws_t = (jnp.where(iota_g == sels[0], w0, 0.0)
            + jnp.where(iota_g == sels[1], w1, 0.0)
            + jnp.where(iota_g == sels[2], w2, 0.0))
    zrow = jnp.zeros((1, NB_BLK), jnp.int32)
    idx_out[0] = jnp.concatenate(
        sels + [zrow, zrow, zrow, zrow, zrow], axis=0)
    zrowf = jnp.zeros((1, NB_BLK), jnp.float32)
    w_out[0] = jnp.concatenate(
        [w0, w1, w2, zrowf, zrowf, zrowf, zrowf, zrowf], axis=0)
    m_blk = lax.dot_general(ws_t, ws_t, (((1,), (1,)), ((), ())),
                            preferred_element_type=jnp.float32)

    @pl.when(nb == 0)
    def _():
        m_out[0] = m_blk

    @pl.when(nb != 0)
    def _():
        m_out[0] += m_blk


def _k1b(xyz_p, cen_p):
    return pl.pallas_call(
        _k1b_body,
        grid=(B, NBB),
        in_specs=[
            pl.BlockSpec((1, NB_BLK, 8), lambda b, nb: (b, nb, 0)),
            pl.BlockSpec((1, G, 8), lambda b, nb: (b, 0, 0)),
        ],
        out_specs=[
            pl.BlockSpec((1, 8, NB_BLK), lambda b, nb: (b, 0, nb)),
            pl.BlockSpec((1, 8, NB_BLK), lambda b, nb: (b, 0, nb)),
            pl.BlockSpec((1, G, G), lambda b, nb: (b, 0, 0)),
        ],
        out_shape=[
            jax.ShapeDtypeStruct((B, 8, N), jnp.int32),
            jax.ShapeDtypeStruct((B, 8, N), jnp.float32),
            jax.ShapeDtypeStruct((B, G, G), jnp.float32),
        ],
        compiler_params=pltpu.CompilerParams(
            dimension_semantics=("arbitrary", "arbitrary")),
    )(xyz_p, cen_p)


def _k1c_body(m_ref, f1_ref, s1, ss1):
    b = pl.program_id(0)
    m = m_ref[0]
    f = f1_ref[0]
    colsum = jnp.sum(m, axis=0, keepdims=True)
    s_blk = jnp.dot(colsum, f, preferred_element_type=jnp.float32)
    mf = jnp.dot(m, f, preferred_element_type=jnp.float32)
    ss_blk = jnp.sum(mf * f, axis=0, keepdims=True)

    @pl.when(b == 0)
    def _():
        s1[...] = s_blk
        ss1[...] = ss_blk

    @pl.when(b != 0)
    def _():
        s1[...] += s_blk
        ss1[...] += ss_blk


def _k1c(m, f1_3d):
    return pl.pallas_call(
        _k1c_body,
        grid=(B,),
        in_specs=[
            pl.BlockSpec((1, G, G), lambda b: (b, 0, 0)),
            pl.BlockSpec((1, G, C), lambda b: (b, 0, 0)),
        ],
        out_specs=[
            pl.BlockSpec((1, C), lambda b: (0, 0)),
            pl.BlockSpec((1, C), lambda b: (0, 0)),
        ],
        out_shape=[
            jax.ShapeDtypeStruct((1, C), jnp.float32),
            jax.ShapeDtypeStruct((1, C), jnp.float32),
        ],
        compiler_params=pltpu.CompilerParams(
            dimension_semantics=("arbitrary",)),
    )(m, f1_3d)


def _sc_body(f1_hbm, idx_hbm, w_hbm, z_hbm,
             f1v, i0v, i1v, i2v, w0v, w1v, w2v, zbuf):
    wid = lax.axis_index("c") * SC_NS + lax.axis_index("s")
    b = wid // 2
    n0 = (wid % 2) * PTS_W
    pltpu.sync_copy(f1_hbm.at[pl.ds(b * G, G)], f1v)
    lanes = lax.iota(jnp.int32, SC_L)

    def chunk_body(t, _):
        s = n0 + t * P_CHUNK
        pltpu.sync_copy(idx_hbm.at[b, 0, pl.ds(s, P_CHUNK)], i0v)
        pltpu.sync_copy(idx_hbm.at[b, 1, pl.ds(s, P_CHUNK)], i1v)
        pltpu.sync_copy(idx_hbm.at[b, 2, pl.ds(s, P_CHUNK)], i2v)
        pltpu.sync_copy(w_hbm.at[b, 0, pl.ds(s, P_CHUNK)], w0v)
        pltpu.sync_copy(w_hbm.at[b, 1, pl.ds(s, P_CHUNK)], w1v)
        pltpu.sync_copy(w_hbm.at[b, 2, pl.ds(s, P_CHUNK)], w2v)

        def pt_body(p, _):
            pvec = jnp.full((SC_L,), p, jnp.int32)
            r0 = plsc.load_gather(i0v, [pvec])
            r1 = plsc.load_gather(i1v, [pvec])
            r2 = plsc.load_gather(i2v, [pvec])
            w0 = plsc.load_gather(w0v, [pvec])
            w1 = plsc.load_gather(w1v, [pvec])
            w2 = plsc.load_gather(w2v, [pvec])
            for j in range(D // SC_L):
                col = lanes + (j * SC_L)
                a0 = plsc.load_gather(f1v, [r0, col])
                a1 = plsc.load_gather(f1v, [r1, col])
                a2 = plsc.load_gather(f1v, [r2, col])
                zbuf[p, pl.ds(j * SC_L, SC_L)] = a0 * w0 + a1 * w1 + a2 * w2
            return 0

        lax.fori_loop(0, P_CHUNK, pt_body, 0)
        pltpu.sync_copy(zbuf, z_hbm.at[pl.ds(wid * PTS_W + t * P_CHUNK,
                                             P_CHUNK)])
        return 0

    lax.fori_loop(0, N_CHUNKS, chunk_body, 0)


def _sc_interp(f1, idx, w):
    mesh = plsc.VectorSubcoreMesh(core_axis_name="c", subcore_axis_name="s")
    run = functools.partial(
        pl.kernel,
        out_type=jax.ShapeDtypeStruct((B * N, C), jnp.float32),
        mesh=mesh,
        compiler_params=pltpu.CompilerParams(needs_layout_passes=False),
        scratch_types=[
            pltpu.VMEM((G, C), jnp.float32),
            pltpu.VMEM((P_CHUNK,), jnp.int32),
            pltpu.VMEM((P_CHUNK,), jnp.int32),
            pltpu.VMEM((P_CHUNK,), jnp.int32),
            pltpu.VMEM((P_CHUNK,), jnp.float32),
            pltpu.VMEM((P_CHUNK,), jnp.float32),
            pltpu.VMEM((P_CHUNK,), jnp.float32),
            pltpu.VMEM((P_CHUNK, C), jnp.float32),
        ],
    )(_sc_body)
    return run(f1, idx, w)


def _k2_body(z, s1, ss1, g1, be1, w2t, b2, y_out, s2, ss2):
    i = pl.program_id(0)
    inv_m = 1.0 / float(B * N)
    mean = s1[...] * inv_m
    var = ss1[...] * inv_m - mean * mean
    a1 = g1[...] * lax.rsqrt(var + 1e-5)
    c1 = be1[...] - mean * a1
    x = jnp.maximum(z[...] * a1 + c1, 0.0)
    y = jnp.dot(x, w2t[...], preferred_element_type=jnp.float32) + b2[...]
    y_out[...] = y
    s_blk = jnp.sum(y, axis=0, keepdims=True)
    ss_blk = jnp.sum(y * y, axis=0, keepdims=True)

    @pl.when(i == 0)
    def _():
        s2[...] = s_blk
        ss2[...] = ss_blk

    @pl.when(i != 0)
    def _():
        s2[...] += s_blk
        ss2[...] += ss_blk


def _k2(z, s1, ss1, g1r, be1r, w2t, b2r):
    vec = pl.BlockSpec((1, C), lambda i: (0, 0))
    return pl.pallas_call(
        _k2_body,
        grid=(NSTEPS,),
        in_specs=[
            pl.BlockSpec((NB_BLK, C), lambda i: (i, 0)),
            vec, vec, vec, vec,
            pl.BlockSpec((C, C), lambda i: (0, 0)),
            vec,
        ],
        out_specs=[
            pl.BlockSpec((NB_BLK, C), lambda i: (i, 0)),
            pl.BlockSpec((1, C), lambda i: (0, 0)),
            pl.BlockSpec((1, C), lambda i: (0, 0)),
        ],
        out_shape=[
            jax.ShapeDtypeStruct((B * N, C), jnp.float32),
            jax.ShapeDtypeStruct((1, C), jnp.float32),
            jax.ShapeDtypeStruct((1, C), jnp.float32),
        ],
        compiler_params=pltpu.CompilerParams(
            dimension_semantics=("arbitrary",)),
    )(z, s1, ss1, g1r, be1r, w2t, b2r)


def _k3_body(y, s2, ss2, g2, be2, out):
    inv_m = 1.0 / float(B * N)
    mean = s2[...] * inv_m
    var = ss2[...] * inv_m - mean * mean
    a2 = g2[...] * lax.rsqrt(var + 1e-5)
    c2 = be2[...] - mean * a2
    out[...] = jnp.maximum(y[...] * a2 + c2, 0.0)


def _k3(y, s2, ss2, g2r, be2r):
    vec = pl.BlockSpec((1, C), lambda i: (0, 0))
    return pl.pallas_call(
        _k3_body,
        grid=(NSTEPS,),
        in_specs=[pl.BlockSpec((NB_BLK, C), lambda i: (i, 0)),
                  vec, vec, vec, vec],
        out_specs=pl.BlockSpec((NB_BLK, C), lambda i: (i, 0)),
        out_shape=jax.ShapeDtypeStruct((B * N, C), jnp.float32),
        compiler_params=pltpu.CompilerParams(
            dimension_semantics=("arbitrary",)),
    )(y, s2, ss2, g2r, be2r)


def kernel(xyz, centers, H4, H8, H12, W1, b1, g1, be1, W2, b2, g2, be2):
    xyz_p = jnp.pad(xyz, ((0, 0), (0, 0), (0, 5)))
    cen_p = jnp.pad(centers, ((0, 0), (0, 0), (0, 5)))
    w1a = W1[:, :D].T
    w1b = W1[:, D:2 * D].T
    w1c = W1[:, 2 * D:].T
    w2t = W2.T
    b1r = b1.reshape(1, C)
    g1r = g1.reshape(1, C)
    be1r = be1.reshape(1, C)
    b2r = b2.reshape(1, C)
    g2r = g2.reshape(1, C)
    be2r = be2.reshape(1, C)

    f1 = _k1a(H4.reshape(B * G, D), H8.reshape(B * G, D),
              H12.reshape(B * G, D), w1a, w1b, w1c, b1r)
    idx, w, m = _k1b(xyz_p, cen_p)
    s1, ss1 = _k1c(m, f1.reshape(B, G, C))
    z = _sc_interp(f1, idx, w)
    y, s2, ss2 = _k2(z, s1, ss1, g1r, be1r, w2t, b2r)
    out = _k3(y, s2, ss2, g2r, be2r)
    return out.reshape(B, N, C)


# speedup vs baseline: 9.7105x; 1.5148x over previous
import functools

import jax
import jax.numpy as jnp
from jax import lax
from jax.experimental import pallas as pl
from jax.experimental.pallas import tpu as pltpu
from jax.experimental.pallas import tpu_sc as plsc

B, N, G, D = 16, 2048, 128, 512
C = 512
NB_BLK = 256
NSTEPS = (B * N) // NB_BLK
NBB = N // NB_BLK

SC_NC, SC_NS, SC_L = 2, 16, 16
NW = SC_NC * SC_NS
PTS_W = (B * N) // NW
P_CHUNK = 32
N_CHUNKS = PTS_W // P_CHUNK


def _k1a_body(h4, h8, h12, w1a, w1b, w1c, b1, f1):
    acc = jnp.dot(h4[...], w1a[...], preferred_element_type=jnp.float32)
    acc += jnp.dot(h8[...], w1b[...], preferred_element_type=jnp.float32)
    acc += jnp.dot(h12[...], w1c[...], preferred_element_type=jnp.float32)
    f1[...] = acc + b1[...]


def _k1a(h4, h8, h12, w1a, w1b, w1c, b1):
    grid = ((B * G) // NB_BLK,)
    blk = pl.BlockSpec((NB_BLK, D), lambda i: (i, 0))
    wblk = pl.BlockSpec((D, C), lambda i: (0, 0))
    return pl.pallas_call(
        _k1a_body,
        grid=grid,
        in_specs=[blk, blk, blk, wblk, wblk, wblk,
                  pl.BlockSpec((1, C), lambda i: (0, 0))],
        out_specs=pl.BlockSpec((NB_BLK, C), lambda i: (i, 0)),
        out_shape=jax.ShapeDtypeStruct((B * G, C), jnp.float32),
        compiler_params=pltpu.CompilerParams(
            dimension_semantics=("arbitrary",)),
    )(h4, h8, h12, w1a, w1b, w1c, b1)


def _k1b_body(xyz, cen, idx_out, w_out, m_out):
    nb = pl.program_id(1)
    x = xyz[0]
    c = cen[0]
    cg2 = jnp.sum(c * c, axis=1, keepdims=True)
    ones_row = jnp.ones((1, 8), jnp.float32)
    xn2 = lax.dot_general(ones_row, x * x,
                          (((1,), (1,)), ((), ())),
                          preferred_element_type=jnp.float32,
                          precision=lax.Precision.HIGHEST)
    cross = lax.dot_general(c.astype(jnp.bfloat16), x.astype(jnp.bfloat16),
                            (((1,), (1,)), ((), ())),
                            preferred_element_type=jnp.float32)
    d = cg2 - 2.0 * cross + xn2
    iota_g = lax.broadcasted_iota(jnp.int32, (G, NB_BLK), 0)
    sels, mins = [], []
    for _ in range(3):
        m = jnp.min(d, axis=0, keepdims=True)
        cand = jnp.where(d == m, iota_g, G)
        sel = jnp.min(cand, axis=0, keepdims=True)
        oh = iota_g == sel
        d = jnp.where(oh, jnp.inf, d)
        sels.append(sel)
        mins.append(m)
    r0 = 1.0 / (mins[0] + 1e-8)
    r1 = 1.0 / (mins[1] + 1e-8)
    r2 = 1.0 / (mins[2] + 1e-8)
    rs = r0 + r1 + r2
    w0, w1, w2 = r0 / rs, r1 / rs, r2 / rs
    ws_t = (jnp.where(iota_g == sels[0], w0, 0.0)
            + jnp.where(iota_g == sels[1], w1, 0.0)
            + jnp.where(iota_g == sels[2], w2, 0.0))
    zrow = jnp.zeros((1, NB_BLK), jnp.int32)
    idx_out[0] = jnp.concatenate(
        sels + [zrow, zrow, zrow, zrow, zrow], axis=0)
    zrowf = jnp.zeros((1, NB_BLK), jnp.float32)
    w_out[0] = jnp.concatenate(
        [w0, w1, w2, zrowf, zrowf, zrowf, zrowf, zrowf], axis=0)
    m_blk = lax.dot_general(ws_t, ws_t, (((1,), (1,)), ((), ())),
                            preferred_element_type=jnp.float32)

    @pl.when(nb == 0)
    def _():
        m_out[0] = m_blk

    @pl.when(nb != 0)
    def _():
        m_out[0] += m_blk


def _k1b(xyz_p, cen_p):
    return pl.pallas_call(
        _k1b_body,
        grid=(B, NBB),
        in_specs=[
            pl.BlockSpec((1, NB_BLK, 8), lambda b, nb: (b, nb, 0)),
            pl.BlockSpec((1, G, 8), lambda b, nb: (b, 0, 0)),
        ],
        out_specs=[
            pl.BlockSpec((1, 8, NB_BLK), lambda b, nb: (b, 0, nb)),
            pl.BlockSpec((1, 8, NB_BLK), lambda b, nb: (b, 0, nb)),
            pl.BlockSpec((1, G, G), lambda b, nb: (b, 0, 0)),
        ],
        out_shape=[
            jax.ShapeDtypeStruct((B, 8, N), jnp.int32),
            jax.ShapeDtypeStruct((B, 8, N), jnp.float32),
            jax.ShapeDtypeStruct((B, G, G), jnp.float32),
        ],
        compiler_params=pltpu.CompilerParams(
            dimension_semantics=("arbitrary", "arbitrary")),
    )(xyz_p, cen_p)


def _k1c_body(m_ref, f1_ref, s1, ss1):
    b = pl.program_id(0)
    m = m_ref[0]
    f = f1_ref[0]
    colsum = jnp.sum(m, axis=0, keepdims=True)
    s_blk = jnp.dot(colsum, f, preferred_element_type=jnp.float32)
    mf = jnp.dot(m, f, preferred_element_type=jnp.float32)
    ss_blk = jnp.sum(mf * f, axis=0, keepdims=True)

    @pl.when(b == 0)
    def _():
        s1[...] = s_blk
        ss1[...] = ss_blk

    @pl.when(b != 0)
    def _():
        s1[...] += s_blk
        ss1[...] += ss_blk


def _k1c(m, f1_3d):
    return pl.pallas_call(
        _k1c_body,
        grid=(B,),
        in_specs=[
            pl.BlockSpec((1, G, G), lambda b: (b, 0, 0)),
            pl.BlockSpec((1, G, C), lambda b: (b, 0, 0)),
        ],
        out_specs=[
            pl.BlockSpec((1, C), lambda b: (0, 0)),
            pl.BlockSpec((1, C), lambda b: (0, 0)),
        ],
        out_shape=[
            jax.ShapeDtypeStruct((1, C), jnp.float32),
            jax.ShapeDtypeStruct((1, C), jnp.float32),
        ],
        compiler_params=pltpu.CompilerParams(
            dimension_semantics=("arbitrary",)),
    )(m, f1_3d)


def _sc_body(f1_hbm, idx_hbm, w_hbm, z_hbm,
             f1v, i0v, i1v, i2v, w0v, w1v, w2v, zbuf):
    wid = lax.axis_index("c") * SC_NS + lax.axis_index("s")
    b = wid // 2
    n0 = (wid % 2) * PTS_W
    pltpu.sync_copy(f1_hbm.at[pl.ds(b * (G * C), G * C)], f1v)
    pltpu.sync_copy(idx_hbm.at[b, 0, pl.ds(n0, PTS_W)], i0v)
    pltpu.sync_copy(idx_hbm.at[b, 1, pl.ds(n0, PTS_W)], i1v)
    pltpu.sync_copy(idx_hbm.at[b, 2, pl.ds(n0, PTS_W)], i2v)
    pltpu.sync_copy(w_hbm.at[b, 0, pl.ds(n0, PTS_W)], w0v)
    pltpu.sync_copy(w_hbm.at[b, 1, pl.ds(n0, PTS_W)], w1v)
    pltpu.sync_copy(w_hbm.at[b, 2, pl.ds(n0, PTS_W)], w2v)
    lanes = lax.iota(jnp.int32, SC_L)

    def chunk_body(t, _):
        @plsc.parallel_loop(0, P_CHUNK)
        def _pt(pp):
            pvec = jnp.full((SC_L,), t * P_CHUNK + pp, jnp.int32)
            b0 = plsc.load_gather(i0v, [pvec]) * C + lanes
            b1 = plsc.load_gather(i1v, [pvec]) * C + lanes
            b2 = plsc.load_gather(i2v, [pvec]) * C + lanes
            w0 = plsc.load_gather(w0v, [pvec])
            w1 = plsc.load_gather(w1v, [pvec])
            w2 = plsc.load_gather(w2v, [pvec])
            for j in range(D // SC_L):
                off = j * SC_L
                a0 = plsc.load_gather(f1v, [b0 + off])
                a1 = plsc.load_gather(f1v, [b1 + off])
                a2 = plsc.load_gather(f1v, [b2 + off])
                zbuf[pp, pl.ds(off, SC_L)] = a0 * w0 + a1 * w1 + a2 * w2

        pltpu.sync_copy(zbuf, z_hbm.at[pl.ds(wid * PTS_W + t * P_CHUNK,
                                             P_CHUNK)])
        return 0

    lax.fori_loop(0, N_CHUNKS, chunk_body, 0)


def _sc_interp(f1, idx, w):
    mesh = plsc.VectorSubcoreMesh(core_axis_name="c", subcore_axis_name="s")
    run = functools.partial(
        pl.kernel,
        out_type=jax.ShapeDtypeStruct((B * N, C), jnp.float32),
        mesh=mesh,
        compiler_params=pltpu.CompilerParams(needs_layout_passes=False),
        scratch_types=[
            pltpu.VMEM((G * C,), jnp.float32),
            pltpu.VMEM((PTS_W,), jnp.int32),
            pltpu.VMEM((PTS_W,), jnp.int32),
            pltpu.VMEM((PTS_W,), jnp.int32),
            pltpu.VMEM((PTS_W,), jnp.float32),
            pltpu.VMEM((PTS_W,), jnp.float32),
            pltpu.VMEM((PTS_W,), jnp.float32),
            pltpu.VMEM((P_CHUNK, C), jnp.float32),
        ],
    )(_sc_body)
    return run(f1.reshape(B * G * C), idx, w)


def _k2_body(z, s1, ss1, g1, be1, w2t, b2, y_out, s2, ss2):
    i = pl.program_id(0)
    inv_m = 1.0 / float(B * N)
    mean = s1[...] * inv_m
    var = ss1[...] * inv_m - mean * mean
    a1 = g1[...] * lax.rsqrt(var + 1e-5)
    c1 = be1[...] - mean * a1
    x = jnp.maximum(z[...] * a1 + c1, 0.0)
    y = jnp.dot(x, w2t[...], preferred_element_type=jnp.float32) + b2[...]
    y_out[...] = y
    s_blk = jnp.sum(y, axis=0, keepdims=True)
    ss_blk = jnp.sum(y * y, axis=0, keepdims=True)

    @pl.when(i == 0)
    def _():
        s2[...] = s_blk
        ss2[...] = ss_blk

    @pl.when(i != 0)
    def _():
        s2[...] += s_blk
        ss2[...] += ss_blk


def _k2(z, s1, ss1, g1r, be1r, w2t, b2r):
    vec = pl.BlockSpec((1, C), lambda i: (0, 0))
    return pl.pallas_call(
        _k2_body,
        grid=(NSTEPS,),
        in_specs=[
            pl.BlockSpec((NB_BLK, C), lambda i: (i, 0)),
            vec, vec, vec, vec,
            pl.BlockSpec((C, C), lambda i: (0, 0)),
            vec,
        ],
        out_specs=[
            pl.BlockSpec((NB_BLK, C), lambda i: (i, 0)),
            pl.BlockSpec((1, C), lambda i: (0, 0)),
            pl.BlockSpec((1, C), lambda i: (0, 0)),
        ],
        out_shape=[
            jax.ShapeDtypeStruct((B * N, C), jnp.float32),
            jax.ShapeDtypeStruct((1, C), jnp.float32),
            jax.ShapeDtypeStruct((1, C), jnp.float32),
        ],
        compiler_params=pltpu.CompilerParams(
            dimension_semantics=("arbitrary",)),
    )(z, s1, ss1, g1r, be1r, w2t, b2r)


def _k3_body(y, s2, ss2, g2, be2, out):
    inv_m = 1.0 / float(B * N)
    mean = s2[...] * inv_m
    var = ss2[...] * inv_m - mean * mean
    a2 = g2[...] * lax.rsqrt(var + 1e-5)
    c2 = be2[...] - mean * a2
    out[...] = jnp.maximum(y[...] * a2 + c2, 0.0)


def _k3(y, s2, ss2, g2r, be2r):
    vec = pl.BlockSpec((1, C), lambda i: (0, 0))
    return pl.pallas_call(
        _k3_body,
        grid=(NSTEPS,),
        in_specs=[pl.BlockSpec((NB_BLK, C), lambda i: (i, 0)),
                  vec, vec, vec, vec],
        out_specs=pl.BlockSpec((NB_BLK, C), lambda i: (i, 0)),
        out_shape=jax.ShapeDtypeStruct((B * N, C), jnp.float32),
        compiler_params=pltpu.CompilerParams(
            dimension_semantics=("arbitrary",)),
    )(y, s2, ss2, g2r, be2r)


def kernel(xyz, centers, H4, H8, H12, W1, b1, g1, be1, W2, b2, g2, be2):
    xyz_p = jnp.pad(xyz, ((0, 0), (0, 0), (0, 5)))
    cen_p = jnp.pad(centers, ((0, 0), (0, 0), (0, 5)))
    w1a = W1[:, :D].T
    w1b = W1[:, D:2 * D].T
    w1c = W1[:, 2 * D:].T
    w2t = W2.T
    b1r = b1.reshape(1, C)
    g1r = g1.reshape(1, C)
    be1r = be1.reshape(1, C)
    b2r = b2.reshape(1, C)
    g2r = g2.reshape(1, C)
    be2r = be2.reshape(1, C)

    f1 = _k1a(H4.reshape(B * G, D), H8.reshape(B * G, D),
              H12.reshape(B * G, D), w1a, w1b, w1c, b1r)
    idx, w, m = _k1b(xyz_p, cen_p)
    s1, ss1 = _k1c(m, f1.reshape(B, G, C))
    z = _sc_interp(f1, idx, w)
    y, s2, ss2 = _k2(z, s1, ss1, g1r, be1r, w2t, b2r)
    out = _k3(y, s2, ss2, g2r, be2r)
    return out.reshape(B, N, C)
